# Initial kernel scaffold; baseline (speedup 1.0000x reference)
import jax
import jax.numpy as jnp
from jax.experimental import pallas as pl


def kernel(graph, h, W_in, b_in, W_convs, W_out, b_out):
    raise NotImplementedError("write your pallas kernel here")


# speedup vs baseline: 5.3050x; 5.3050x over previous
import functools
import math

import jax
import jax.numpy as jnp
from jax import lax
from jax.experimental import pallas as pl
from jax.experimental.pallas import tpu as pltpu, tpu_sc as plsc

N_NODES = 10000
N_EDGES = 320000
NFEAT = 128
NHID = 128
NCLASS = 40
NUM_LAYERS = 4
ALPHA = 0.1
LAM = 0.5

NC = 2
NS = 16
NW = NC * NS

N_PAD = 10240
E_PAD = NW * 80 * 128
CHUNK = 128
STEPS = E_PAD // (NW * CHUNK)
ROWS_PER_TILE = N_PAD // NW
ROWS_PER_SUB = N_PAD // NS

_MESH = plsc.VectorSubcoreMesh(
    core_axis_name="c", subcore_axis_name="s", num_cores=NC, num_subcores=NS)


def _zero_span(zbuf_v, acc_sh, base, rows, zrows, minor):
    z16 = jnp.zeros((16,), jnp.float32)
    for i in range(zrows):
        for c in range(minor // 16):
            zbuf_v[i, pl.ds(c * 16, 16)] = z16
    def body(k, _):
        pltpu.sync_copy(zbuf_v, acc_sh.at[pl.ds(base + k * zrows, zrows)])
        return 0
    lax.fori_loop(0, rows // zrows, body, 0)


@functools.partial(
    pl.kernel,
    out_type=jax.ShapeDtypeStruct((NC, N_PAD, 16), jnp.float32),
    mesh=_MESH,
    scratch_types=[
        pltpu.VMEM((CHUNK,), jnp.int32),
        pltpu.VMEM((CHUNK, 16), jnp.float32),
        pltpu.VMEM((16, 16), jnp.float32),
        pltpu.VMEM_SHARED((N_PAD, 16), jnp.float32),
    ],
)
def _sc_degree(dst_hbm, out_hbm, idx_v, ones_v, zbuf_v, acc_sh):
    c = lax.axis_index("c")
    s = lax.axis_index("s")
    wid = s * NC + c
    one16 = jnp.full((16,), 1.0, jnp.float32)
    for i in range(CHUNK):
        ones_v[i, :] = one16
    _zero_span(zbuf_v, acc_sh, s * ROWS_PER_TILE, ROWS_PER_TILE, 16, 16)
    plsc.subcore_barrier()

    def body(j, _):
        pltpu.sync_copy(dst_hbm.at[wid, j], idx_v)
        pltpu.sync_copy(ones_v, acc_sh.at[idx_v], add=True)
        return 0
    lax.fori_loop(0, STEPS, body, 0)

    plsc.subcore_barrier()
    base = s * ROWS_PER_SUB
    pltpu.sync_copy(acc_sh.at[pl.ds(base, ROWS_PER_SUB)],
                    out_hbm.at[c, pl.ds(base, ROWS_PER_SUB)])


@functools.partial(
    pl.kernel,
    out_type=jax.ShapeDtypeStruct((NC, N_PAD, NHID), jnp.float32),
    mesh=_MESH,
    scratch_types=[
        pltpu.VMEM((CHUNK,), jnp.int32),
        pltpu.VMEM((CHUNK,), jnp.int32),
        pltpu.VMEM((CHUNK, NHID), jnp.float32),
        pltpu.VMEM((16, NHID), jnp.float32),
        pltpu.VMEM_SHARED((N_PAD, NHID), jnp.float32),
        pltpu.SemaphoreType.DMA,
    ],
)
def _sc_propagate(g_hbm, src_hbm, dst_hbm, out_hbm,
                  src_v, dst_v, rows_v, zbuf_v, acc_sh, sem):
    c = lax.axis_index("c")
    s = lax.axis_index("s")
    wid = s * NC + c
    _zero_span(zbuf_v, acc_sh, s * ROWS_PER_TILE, ROWS_PER_TILE, 16, NHID)
    plsc.subcore_barrier()

    def body(j, _):
        pltpu.sync_copy(src_hbm.at[wid, j], src_v)
        pltpu.sync_copy(dst_hbm.at[wid, j], dst_v)
        pltpu.async_copy(g_hbm.at[src_v], rows_v, sem).wait()
        pltpu.sync_copy(rows_v, acc_sh.at[dst_v], add=True)
        return 0
    lax.fori_loop(0, STEPS, body, 0)

    plsc.subcore_barrier()
    base = s * ROWS_PER_SUB
    pltpu.sync_copy(acc_sh.at[pl.ds(base, ROWS_PER_SUB)],
                    out_hbm.at[c, pl.ds(base, ROWS_PER_SUB)])


BLK = 512
GRID = N_PAD // BLK

_row_spec = pl.BlockSpec((BLK, NHID), lambda i: (i, 0))
_par_spec = pl.BlockSpec((NC, BLK, 16), lambda i: (0, i, 0))
_parh_spec = pl.BlockSpec((NC, BLK, NHID), lambda i: (0, i, 0))
_w_spec = pl.BlockSpec((NHID, NHID), lambda i: (0, 0))
_b_spec = pl.BlockSpec((1, NHID), lambda i: (0, 0))


def _tc_prep_body(hp_ref, win_ref, bin_ref, degp_ref,
                  dinv_ref, h0_ref, g_ref):
    i = pl.program_id(0)
    cnt = (jnp.sum(degp_ref[0], axis=1, keepdims=True)
           + jnp.sum(degp_ref[1], axis=1, keepdims=True)) * (1.0 / 16.0)
    deg = cnt + 1.0
    rows = lax.broadcasted_iota(jnp.int32, (BLK, 1), 0) + i * BLK
    dinv = jnp.where(rows < N_NODES, lax.rsqrt(deg), 0.0)
    h1 = jnp.dot(hp_ref[...], win_ref[...],
                 preferred_element_type=jnp.float32) + bin_ref[...]
    h1 = jnp.maximum(h1, 0.0)
    dinv_ref[...] = jnp.broadcast_to(dinv, (BLK, NHID))
    h0_ref[...] = h1
    g_ref[...] = dinv * h1


def _tc_prep(h_pad, w_in, b_in, degp):
    return pl.pallas_call(
        _tc_prep_body,
        grid=(GRID,),
        in_specs=[_row_spec, _w_spec, _b_spec, _par_spec],
        out_specs=[_row_spec, _row_spec, _row_spec],
        out_shape=[jax.ShapeDtypeStruct((N_PAD, NHID), jnp.float32)] * 3,
    )(h_pad, w_in, b_in, degp)


def _make_combine(beta, relu):
    def body(p_ref, g_ref, h0_ref, dinv_ref, w_ref, gout_ref):
        ah = dinv_ref[...] * (p_ref[0] + p_ref[1] + g_ref[...])
        sup = (1.0 - ALPHA) * ah + ALPHA * h0_ref[...]
        h = (1.0 - beta) * sup + beta * jnp.dot(
            sup, w_ref[...], preferred_element_type=jnp.float32)
        if relu:
            h = jnp.maximum(h, 0.0)
        gout_ref[...] = dinv_ref[...] * h

    def run(p, g, h0, dinv2d, w):
        return pl.pallas_call(
            body,
            grid=(GRID,),
            in_specs=[_parh_spec, _row_spec, _row_spec, _row_spec, _w_spec],
            out_specs=_row_spec,
            out_shape=jax.ShapeDtypeStruct((N_PAD, NHID), jnp.float32),
        )(p, g, h0, dinv2d, w)
    return run


def _make_final(beta):
    def body(p_ref, g_ref, h0_ref, dinv_ref, w_ref, wout_ref, bout_ref,
             out_ref):
        ah = dinv_ref[...] * (p_ref[0] + p_ref[1] + g_ref[...])
        sup = (1.0 - ALPHA) * ah + ALPHA * h0_ref[...]
        h = (1.0 - beta) * sup + beta * jnp.dot(
            sup, w_ref[...], preferred_element_type=jnp.float32)
        logits = jnp.dot(h, wout_ref[...],
                         preferred_element_type=jnp.float32) + bout_ref[...]
        col = lax.broadcasted_iota(jnp.int32, (BLK, NHID), 1)
        logits = jnp.where(col < NCLASS, logits, -1e30)
        m = jnp.max(logits, axis=1, keepdims=True)
        lse = jnp.log(jnp.sum(jnp.exp(logits - m), axis=1, keepdims=True)) + m
        out_ref[...] = logits - lse

    def run(p, g, h0, dinv2d, w, w_out, b_out):
        return pl.pallas_call(
            body,
            grid=(GRID,),
            in_specs=[_parh_spec, _row_spec, _row_spec, _row_spec, _w_spec,
                      _w_spec, _b_spec],
            out_specs=_row_spec,
            out_shape=jax.ShapeDtypeStruct((N_PAD, NHID), jnp.float32),
        )(p, g, h0, dinv2d, w, w_out, b_out)
    return run


@jax.jit
def kernel(graph, h, W_in, b_in, W_convs, W_out, b_out):
    graph = graph.astype(jnp.int32)
    pad = jnp.full((E_PAD - N_EDGES,), N_NODES, jnp.int32)
    src = jnp.concatenate([graph[0], pad]).reshape(NW, STEPS, CHUNK)
    dst = jnp.concatenate([graph[1], pad]).reshape(NW, STEPS, CHUNK)

    h_pad = jnp.pad(h, ((0, N_PAD - N_NODES), (0, 0)))
    w_out_pad = jnp.pad(W_out, ((0, 0), (0, NHID - NCLASS)))
    b_out_pad = jnp.pad(b_out, ((0, NHID - NCLASS),)).reshape(1, NHID)
    b_in2 = b_in.reshape(1, NHID)

    degp = _sc_degree(dst)
    dinv2d, h0, g = _tc_prep(h_pad, W_in, b_in2, degp)

    for l in range(1, NUM_LAYERS + 1):
        beta = math.log(LAM / l + 1.0)
        p = _sc_propagate(g, src, dst)
        if l < NUM_LAYERS:
            g = _make_combine(beta, relu=(l == 1))(p, g, h0, dinv2d,
                                                   W_convs[l - 1])
        else:
            out = _make_final(beta)(p, g, h0, dinv2d, W_convs[l - 1],
                                    w_out_pad, b_out_pad)
    return out[:N_NODES, :NCLASS]


# speedup vs baseline: 5.8381x; 1.1005x over previous
import functools
import math

import jax
import jax.numpy as jnp
from jax import lax
from jax.experimental import pallas as pl
from jax.experimental.pallas import tpu as pltpu, tpu_sc as plsc

N_NODES = 10000
N_EDGES = 320000
NFEAT = 128
NHID = 128
NCLASS = 40
NUM_LAYERS = 4
ALPHA = 0.1
LAM = 0.5

NC = 2
NS = 16
NW = NC * NS

N_PAD = 10240
E_PAD = NW * 80 * 128
CHUNK = 128
STEPS = E_PAD // (NW * CHUNK)
NBUF = 2
ROWS_PER_TILE = N_PAD // NW
ROWS_PER_SUB = N_PAD // NS

_MESH = plsc.VectorSubcoreMesh(
    core_axis_name="c", subcore_axis_name="s", num_cores=NC, num_subcores=NS)


def _zero_span(zbuf_v, acc_sh, base, rows, zrows, minor):
    z16 = jnp.zeros((16,), jnp.float32)
    for i in range(zrows):
        for c in range(minor // 16):
            zbuf_v[i, pl.ds(c * 16, 16)] = z16
    def body(k, _):
        pltpu.sync_copy(zbuf_v, acc_sh.at[pl.ds(base + k * zrows, zrows)])
        return 0
    lax.fori_loop(0, rows // zrows, body, 0)


@functools.partial(
    pl.kernel,
    out_type=jax.ShapeDtypeStruct((NC, N_PAD, 16), jnp.float32),
    mesh=_MESH,
    scratch_types=[
        pltpu.VMEM((CHUNK,), jnp.int32),
        pltpu.VMEM((CHUNK, 16), jnp.float32),
        pltpu.VMEM((16, 16), jnp.float32),
        pltpu.VMEM_SHARED((N_PAD, 16), jnp.float32),
    ],
)
def _sc_degree(dst_hbm, out_hbm, idx_v, ones_v, zbuf_v, acc_sh):
    c = lax.axis_index("c")
    s = lax.axis_index("s")
    wid = s * NC + c
    one16 = jnp.full((16,), 1.0, jnp.float32)
    for i in range(CHUNK):
        ones_v[i, :] = one16
    _zero_span(zbuf_v, acc_sh, s * ROWS_PER_TILE, ROWS_PER_TILE, 16, 16)
    plsc.subcore_barrier()

    def body(j, _):
        pltpu.sync_copy(dst_hbm.at[wid, j], idx_v)
        pltpu.sync_copy(ones_v, acc_sh.at[idx_v], add=True)
        return 0
    lax.fori_loop(0, STEPS, body, 0)

    plsc.subcore_barrier()
    base = s * ROWS_PER_SUB
    pltpu.sync_copy(acc_sh.at[pl.ds(base, ROWS_PER_SUB)],
                    out_hbm.at[c, pl.ds(base, ROWS_PER_SUB)])


@functools.partial(
    pl.kernel,
    out_type=jax.ShapeDtypeStruct((NC, N_PAD, NHID), jnp.float32),
    mesh=_MESH,
    scratch_types=(
        [pltpu.VMEM((CHUNK,), jnp.int32)] * (2 * NBUF)
        + [pltpu.VMEM((CHUNK, NHID), jnp.float32)] * NBUF
        + [pltpu.VMEM((16, NHID), jnp.float32),
           pltpu.VMEM_SHARED((N_PAD, NHID), jnp.float32),
           pltpu.SemaphoreType.DMA]
        + [pltpu.SemaphoreType.DMA] * NBUF
    ),
)
def _sc_propagate(g_hbm, src_hbm, dst_hbm, out_hbm, *refs):
    src_v = refs[0:NBUF]
    dst_v = refs[NBUF:2 * NBUF]
    rows_v = refs[2 * NBUF:3 * NBUF]
    zbuf_v = refs[3 * NBUF]
    acc_sh = refs[3 * NBUF + 1]
    sem_i = refs[3 * NBUF + 2]
    sems = refs[3 * NBUF + 3:]
    c = lax.axis_index("c")
    s = lax.axis_index("s")
    wid = s * NC + c
    _zero_span(zbuf_v, acc_sh, s * ROWS_PER_TILE, ROWS_PER_TILE, 16, NHID)
    plsc.subcore_barrier()

    def body(k, _):
        j0 = NBUF * k
        idx_cps = []
        for b in range(NBUF):
            idx_cps.append(
                pltpu.async_copy(src_hbm.at[wid, j0 + b], src_v[b], sem_i))
            idx_cps.append(
                pltpu.async_copy(dst_hbm.at[wid, j0 + b], dst_v[b], sem_i))
        for cp in idx_cps:
            cp.wait()
        row_cps = [
            pltpu.async_copy(g_hbm.at[src_v[b]], rows_v[b], sems[b])
            for b in range(NBUF)]
        for b in range(NBUF):
            row_cps[b].wait()
            pltpu.sync_copy(rows_v[b], acc_sh.at[dst_v[b]], add=True)
        return 0
    lax.fori_loop(0, STEPS // NBUF, body, 0)

    plsc.subcore_barrier()
    base = s * ROWS_PER_SUB
    pltpu.sync_copy(acc_sh.at[pl.ds(base, ROWS_PER_SUB)],
                    out_hbm.at[c, pl.ds(base, ROWS_PER_SUB)])


BLK = 512
GRID = N_PAD // BLK

_row_spec = pl.BlockSpec((BLK, NHID), lambda i: (i, 0))
_par_spec = pl.BlockSpec((NC, BLK, 16), lambda i: (0, i, 0))
_parh_spec = pl.BlockSpec((NC, BLK, NHID), lambda i: (0, i, 0))
_w_spec = pl.BlockSpec((NHID, NHID), lambda i: (0, 0))
_b_spec = pl.BlockSpec((1, NHID), lambda i: (0, 0))


def _tc_prep_body(hp_ref, win_ref, bin_ref, degp_ref,
                  dinv_ref, h0_ref, g_ref):
    i = pl.program_id(0)
    cnt = (jnp.sum(degp_ref[0], axis=1, keepdims=True)
           + jnp.sum(degp_ref[1], axis=1, keepdims=True)) * (1.0 / 16.0)
    deg = cnt + 1.0
    rows = lax.broadcasted_iota(jnp.int32, (BLK, 1), 0) + i * BLK
    dinv = jnp.where(rows < N_NODES, lax.rsqrt(deg), 0.0)
    h1 = jnp.dot(hp_ref[...], win_ref[...],
                 preferred_element_type=jnp.float32) + bin_ref[...]
    h1 = jnp.maximum(h1, 0.0)
    dinv_ref[...] = jnp.broadcast_to(dinv, (BLK, NHID))
    h0_ref[...] = h1
    g_ref[...] = dinv * h1


def _tc_prep(h_pad, w_in, b_in, degp):
    return pl.pallas_call(
        _tc_prep_body,
        grid=(GRID,),
        in_specs=[_row_spec, _w_spec, _b_spec, _par_spec],
        out_specs=[_row_spec, _row_spec, _row_spec],
        out_shape=[jax.ShapeDtypeStruct((N_PAD, NHID), jnp.float32)] * 3,
    )(h_pad, w_in, b_in, degp)


def _make_combine(beta, relu):
    def body(p_ref, g_ref, h0_ref, dinv_ref, w_ref, gout_ref):
        ah = dinv_ref[...] * (p_ref[0] + p_ref[1] + g_ref[...])
        sup = (1.0 - ALPHA) * ah + ALPHA * h0_ref[...]
        h = (1.0 - beta) * sup + beta * jnp.dot(
            sup, w_ref[...], preferred_element_type=jnp.float32)
        if relu:
            h = jnp.maximum(h, 0.0)
        gout_ref[...] = dinv_ref[...] * h

    def run(p, g, h0, dinv2d, w):
        return pl.pallas_call(
            body,
            grid=(GRID,),
            in_specs=[_parh_spec, _row_spec, _row_spec, _row_spec, _w_spec],
            out_specs=_row_spec,
            out_shape=jax.ShapeDtypeStruct((N_PAD, NHID), jnp.float32),
        )(p, g, h0, dinv2d, w)
    return run


def _make_final(beta):
    def body(p_ref, g_ref, h0_ref, dinv_ref, w_ref, wout_ref, bout_ref,
             out_ref):
        ah = dinv_ref[...] * (p_ref[0] + p_ref[1] + g_ref[...])
        sup = (1.0 - ALPHA) * ah + ALPHA * h0_ref[...]
        h = (1.0 - beta) * sup + beta * jnp.dot(
            sup, w_ref[...], preferred_element_type=jnp.float32)
        logits = jnp.dot(h, wout_ref[...],
                         preferred_element_type=jnp.float32) + bout_ref[...]
        col = lax.broadcasted_iota(jnp.int32, (BLK, NHID), 1)
        logits = jnp.where(col < NCLASS, logits, -1e30)
        m = jnp.max(logits, axis=1, keepdims=True)
        lse = jnp.log(jnp.sum(jnp.exp(logits - m), axis=1, keepdims=True)) + m
        out_ref[...] = logits - lse

    def run(p, g, h0, dinv2d, w, w_out, b_out):
        return pl.pallas_call(
            body,
            grid=(GRID,),
            in_specs=[_parh_spec, _row_spec, _row_spec, _row_spec, _w_spec,
                      _w_spec, _b_spec],
            out_specs=_row_spec,
            out_shape=jax.ShapeDtypeStruct((N_PAD, NHID), jnp.float32),
        )(p, g, h0, dinv2d, w, w_out, b_out)
    return run


@jax.jit
def kernel(graph, h, W_in, b_in, W_convs, W_out, b_out):
    graph = graph.astype(jnp.int32)
    pad = jnp.full((E_PAD - N_EDGES,), N_NODES, jnp.int32)
    src = jnp.concatenate([graph[0], pad]).reshape(NW, STEPS, CHUNK)
    dst = jnp.concatenate([graph[1], pad]).reshape(NW, STEPS, CHUNK)

    h_pad = jnp.pad(h, ((0, N_PAD - N_NODES), (0, 0)))
    w_out_pad = jnp.pad(W_out, ((0, 0), (0, NHID - NCLASS)))
    b_out_pad = jnp.pad(b_out, ((0, NHID - NCLASS),)).reshape(1, NHID)
    b_in2 = b_in.reshape(1, NHID)

    degp = _sc_degree(dst)
    dinv2d, h0, g = _tc_prep(h_pad, W_in, b_in2, degp)

    for l in range(1, NUM_LAYERS + 1):
        beta = math.log(LAM / l + 1.0)
        p = _sc_propagate(g, src, dst)
        if l < NUM_LAYERS:
            g = _make_combine(beta, relu=(l == 1))(p, g, h0, dinv2d,
                                                   W_convs[l - 1])
        else:
            out = _make_final(beta)(p, g, h0, dinv2d, W_convs[l - 1],
                                    w_out_pad, b_out_pad)
    return out[:N_NODES, :NCLASS]


# speedup vs baseline: 6.6175x; 1.1335x over previous
import functools
import math

import jax
import jax.numpy as jnp
from jax import lax
from jax.experimental import pallas as pl
from jax.experimental.pallas import tpu as pltpu, tpu_sc as plsc

N_NODES = 10000
N_EDGES = 320000
NFEAT = 128
NHID = 128
NCLASS = 40
NUM_LAYERS = 4
ALPHA = 0.1
LAM = 0.5

NC = 2
NS = 16
NW = NC * NS

N_PAD = 10240
E_PAD = NW * 80 * 128
CHUNK = 128
STEPS = E_PAD // (NW * CHUNK)
NBUF = 2

N_CHUNKS = E_PAD // CHUNK
STEPS_FAST = 126
STEPS_SLOW = (N_CHUNKS - 16 * STEPS_FAST) // 16
CHUNKS_FAST = 16 * STEPS_FAST
ROWS_PER_TILE = N_PAD // NW
ROWS_PER_SUB = N_PAD // NS

_MESH = plsc.VectorSubcoreMesh(
    core_axis_name="c", subcore_axis_name="s", num_cores=NC, num_subcores=NS)


@functools.partial(
    pl.kernel,
    out_type=jax.ShapeDtypeStruct((NC, N_PAD, 16), jnp.float32),
    mesh=_MESH,
    scratch_types=[
        pltpu.VMEM((CHUNK,), jnp.int32),
        pltpu.VMEM((CHUNK, 16), jnp.float32),
        pltpu.VMEM_SHARED((N_PAD, 16), jnp.float32),
    ],
)
def _sc_degree(dst_hbm, zeros_hbm, out_hbm, idx_v, ones_v, acc_sh):
    c = lax.axis_index("c")
    s = lax.axis_index("s")
    wid = s * NC + c
    one16 = jnp.full((16,), 1.0, jnp.float32)
    for i in range(CHUNK):
        ones_v[i, :] = one16
    base0 = s * ROWS_PER_SUB
    pltpu.sync_copy(zeros_hbm.at[pl.ds(base0, ROWS_PER_SUB)],
                    acc_sh.at[pl.ds(base0, ROWS_PER_SUB)])
    plsc.subcore_barrier()

    def body(j, _):
        pltpu.sync_copy(dst_hbm.at[wid * STEPS + j], idx_v)
        pltpu.sync_copy(ones_v, acc_sh.at[idx_v], add=True)
        return 0
    lax.fori_loop(0, STEPS, body, 0)

    plsc.subcore_barrier()
    base = s * ROWS_PER_SUB
    pltpu.sync_copy(acc_sh.at[pl.ds(base, ROWS_PER_SUB)],
                    out_hbm.at[c, pl.ds(base, ROWS_PER_SUB)])


@functools.partial(
    pl.kernel,
    out_type=jax.ShapeDtypeStruct((NC, N_PAD, NHID), jnp.float32),
    mesh=_MESH,
    scratch_types=(
        [pltpu.VMEM((CHUNK,), jnp.int32)] * (2 * NBUF)
        + [pltpu.VMEM((CHUNK, NHID), jnp.float32)] * NBUF
        + [pltpu.VMEM_SHARED((N_PAD, NHID), jnp.float32),
           pltpu.SemaphoreType.DMA]
        + [pltpu.SemaphoreType.DMA] * NBUF
    ),
)
def _sc_propagate(g_hbm, src_hbm, dst_hbm, zeros_hbm, out_hbm, *refs):
    src_v = refs[0:NBUF]
    dst_v = refs[NBUF:2 * NBUF]
    rows_v = refs[2 * NBUF:3 * NBUF]
    acc_sh = refs[3 * NBUF]
    sem_i = refs[3 * NBUF + 1]
    sems = refs[3 * NBUF + 2:]
    c = lax.axis_index("c")
    s = lax.axis_index("s")
    base0 = s * ROWS_PER_SUB
    pltpu.sync_copy(zeros_hbm.at[pl.ds(base0, ROWS_PER_SUB)],
                    acc_sh.at[pl.ds(base0, ROWS_PER_SUB)])
    plsc.subcore_barrier()

    def make_body(base):
        def body(k, _):
            j0 = base + NBUF * k
            idx_cps = []
            for b in range(NBUF):
                idx_cps.append(
                    pltpu.async_copy(src_hbm.at[j0 + b], src_v[b], sem_i))
                idx_cps.append(
                    pltpu.async_copy(dst_hbm.at[j0 + b], dst_v[b], sem_i))
            for cp in idx_cps:
                cp.wait()
            row_cps = [
                pltpu.async_copy(g_hbm.at[src_v[b]], rows_v[b], sems[b])
                for b in range(NBUF)]
            for b in range(NBUF):
                row_cps[b].wait()
                pltpu.sync_copy(rows_v[b], acc_sh.at[dst_v[b]], add=True)
            return 0
        return body

    @pl.when(c == 0)
    def _():
        lax.fori_loop(0, STEPS_FAST // NBUF,
                      make_body(s * STEPS_FAST), 0)

    @pl.when(c == 1)
    def _():
        lax.fori_loop(0, STEPS_SLOW // NBUF,
                      make_body(CHUNKS_FAST + s * STEPS_SLOW), 0)

    plsc.subcore_barrier()
    base = s * ROWS_PER_SUB
    pltpu.sync_copy(acc_sh.at[pl.ds(base, ROWS_PER_SUB)],
                    out_hbm.at[c, pl.ds(base, ROWS_PER_SUB)])


BLK = 512
GRID = N_PAD // BLK

_row_spec = pl.BlockSpec((BLK, NHID), lambda i: (i, 0))
_par_spec = pl.BlockSpec((NC, BLK, 16), lambda i: (0, i, 0))
_parh_spec = pl.BlockSpec((NC, BLK, NHID), lambda i: (0, i, 0))
_w_spec = pl.BlockSpec((NHID, NHID), lambda i: (0, 0))
_b_spec = pl.BlockSpec((1, NHID), lambda i: (0, 0))


def _tc_prep_body(hp_ref, win_ref, bin_ref, degp_ref,
                  dinv_ref, h0_ref, g_ref):
    i = pl.program_id(0)
    cnt = (jnp.sum(degp_ref[0], axis=1, keepdims=True)
           + jnp.sum(degp_ref[1], axis=1, keepdims=True)) * (1.0 / 16.0)
    deg = cnt + 1.0
    rows = lax.broadcasted_iota(jnp.int32, (BLK, 1), 0) + i * BLK
    dinv = jnp.where(rows < N_NODES, lax.rsqrt(deg), 0.0)
    h1 = jnp.dot(hp_ref[...], win_ref[...],
                 preferred_element_type=jnp.float32) + bin_ref[...]
    h1 = jnp.maximum(h1, 0.0)
    dinv_ref[...] = jnp.broadcast_to(dinv, (BLK, NHID))
    h0_ref[...] = h1
    g_ref[...] = dinv * h1


def _tc_prep(h_pad, w_in, b_in, degp):
    return pl.pallas_call(
        _tc_prep_body,
        grid=(GRID,),
        in_specs=[_row_spec, _w_spec, _b_spec, _par_spec],
        out_specs=[_row_spec, _row_spec, _row_spec],
        out_shape=[jax.ShapeDtypeStruct((N_PAD, NHID), jnp.float32)] * 3,
    )(h_pad, w_in, b_in, degp)


def _make_combine(beta, relu):
    def body(p_ref, g_ref, h0_ref, dinv_ref, w_ref, gout_ref):
        ah = dinv_ref[...] * (p_ref[0] + p_ref[1] + g_ref[...])
        sup = (1.0 - ALPHA) * ah + ALPHA * h0_ref[...]
        h = (1.0 - beta) * sup + beta * jnp.dot(
            sup, w_ref[...], preferred_element_type=jnp.float32)
        if relu:
            h = jnp.maximum(h, 0.0)
        gout_ref[...] = dinv_ref[...] * h

    def run(p, g, h0, dinv2d, w):
        return pl.pallas_call(
            body,
            grid=(GRID,),
            in_specs=[_parh_spec, _row_spec, _row_spec, _row_spec, _w_spec],
            out_specs=_row_spec,
            out_shape=jax.ShapeDtypeStruct((N_PAD, NHID), jnp.float32),
        )(p, g, h0, dinv2d, w)
    return run


def _make_final(beta):
    def body(p_ref, g_ref, h0_ref, dinv_ref, w_ref, wout_ref, bout_ref,
             out_ref):
        ah = dinv_ref[...] * (p_ref[0] + p_ref[1] + g_ref[...])
        sup = (1.0 - ALPHA) * ah + ALPHA * h0_ref[...]
        h = (1.0 - beta) * sup + beta * jnp.dot(
            sup, w_ref[...], preferred_element_type=jnp.float32)
        logits = jnp.dot(h, wout_ref[...],
                         preferred_element_type=jnp.float32) + bout_ref[...]
        col = lax.broadcasted_iota(jnp.int32, (BLK, NHID), 1)
        logits = jnp.where(col < NCLASS, logits, -1e30)
        m = jnp.max(logits, axis=1, keepdims=True)
        lse = jnp.log(jnp.sum(jnp.exp(logits - m), axis=1, keepdims=True)) + m
        out_ref[...] = logits - lse

    def run(p, g, h0, dinv2d, w, w_out, b_out):
        return pl.pallas_call(
            body,
            grid=(GRID,),
            in_specs=[_parh_spec, _row_spec, _row_spec, _row_spec, _w_spec,
                      _w_spec, _b_spec],
            out_specs=_row_spec,
            out_shape=jax.ShapeDtypeStruct((N_PAD, NHID), jnp.float32),
        )(p, g, h0, dinv2d, w, w_out, b_out)
    return run


@jax.jit
def kernel(graph, h, W_in, b_in, W_convs, W_out, b_out):
    graph = graph.astype(jnp.int32)
    pad = jnp.full((E_PAD - N_EDGES,), N_NODES, jnp.int32)
    src = jnp.concatenate([graph[0], pad]).reshape(N_CHUNKS, CHUNK)
    dst = jnp.concatenate([graph[1], pad]).reshape(N_CHUNKS, CHUNK)

    h_pad = jnp.pad(h, ((0, N_PAD - N_NODES), (0, 0)))
    w_out_pad = jnp.pad(W_out, ((0, 0), (0, NHID - NCLASS)))
    b_out_pad = jnp.pad(b_out, ((0, NHID - NCLASS),)).reshape(1, NHID)
    b_in2 = b_in.reshape(1, NHID)

    zeros16 = jnp.zeros((N_PAD, 16), jnp.float32)
    zerosh = jnp.zeros((N_PAD, NHID), jnp.float32)
    degp = _sc_degree(dst, zeros16)
    dinv2d, h0, g = _tc_prep(h_pad, W_in, b_in2, degp)

    for l in range(1, NUM_LAYERS + 1):
        beta = math.log(LAM / l + 1.0)
        p = _sc_propagate(g, src, dst, zerosh)
        if l < NUM_LAYERS:
            g = _make_combine(beta, relu=(l == 1))(p, g, h0, dinv2d,
                                                   W_convs[l - 1])
        else:
            out = _make_final(beta)(p, g, h0, dinv2d, W_convs[l - 1],
                                    w_out_pad, b_out_pad)
    return out[:N_NODES, :NCLASS]


# speedup vs baseline: 7.9380x; 1.1995x over previous
import functools
import math

import jax
import jax.numpy as jnp
from jax import lax
from jax.experimental import pallas as pl
from jax.experimental.pallas import tpu as pltpu, tpu_sc as plsc

N_NODES = 10000
N_EDGES = 320000
NFEAT = 128
NHID = 128
NCLASS = 40
NUM_LAYERS = 4
ALPHA = 0.1
LAM = 0.5

NC = 2
NS = 16
NW = NC * NS

N_PAD = 10240
E_PAD = NW * 80 * 128
CHUNK = 128
STEPS = E_PAD // (NW * CHUNK)
NBUF = 2

N_CHUNKS = E_PAD // CHUNK
STEPS_FAST = 142
STEPS_SLOW = (N_CHUNKS - 16 * STEPS_FAST) // 16
CHUNKS_FAST = 16 * STEPS_FAST
ROWS_PER_TILE = N_PAD // NW
ROWS_PER_SUB = N_PAD // NS

_MESH = plsc.VectorSubcoreMesh(
    core_axis_name="c", subcore_axis_name="s", num_cores=NC, num_subcores=NS)


@functools.partial(
    pl.kernel,
    out_type=jax.ShapeDtypeStruct((NC, N_PAD, 16), jnp.float32),
    mesh=_MESH,
    scratch_types=[
        pltpu.VMEM((CHUNK,), jnp.int32),
        pltpu.VMEM((CHUNK, 16), jnp.float32),
        pltpu.VMEM_SHARED((N_PAD, 16), jnp.float32),
    ],
)
def _sc_degree(dst_hbm, zeros_hbm, out_hbm, idx_v, ones_v, acc_sh):
    c = lax.axis_index("c")
    s = lax.axis_index("s")
    wid = s * NC + c
    one16 = jnp.full((16,), 1.0, jnp.float32)
    for i in range(CHUNK):
        ones_v[i, :] = one16
    base0 = s * ROWS_PER_SUB
    pltpu.sync_copy(zeros_hbm.at[pl.ds(base0, ROWS_PER_SUB)],
                    acc_sh.at[pl.ds(base0, ROWS_PER_SUB)])
    plsc.subcore_barrier()

    def body(j, _):
        pltpu.sync_copy(dst_hbm.at[wid * STEPS + j], idx_v)
        pltpu.sync_copy(ones_v, acc_sh.at[idx_v], add=True)
        return 0
    lax.fori_loop(0, STEPS, body, 0)

    plsc.subcore_barrier()
    base = s * ROWS_PER_SUB
    pltpu.sync_copy(acc_sh.at[pl.ds(base, ROWS_PER_SUB)],
                    out_hbm.at[c, pl.ds(base, ROWS_PER_SUB)])


@functools.partial(
    pl.kernel,
    out_type=jax.ShapeDtypeStruct((NC, N_PAD, NHID), jnp.float32),
    mesh=_MESH,
    scratch_types=(
        [pltpu.VMEM((CHUNK,), jnp.int32)] * (2 * NBUF)
        + [pltpu.VMEM((CHUNK, NHID), jnp.float32)] * NBUF
        + [pltpu.VMEM_SHARED((N_PAD, NHID), jnp.float32),
           pltpu.SemaphoreType.DMA]
        + [pltpu.SemaphoreType.DMA] * NBUF
    ),
)
def _sc_propagate(g_hbm, src_hbm, dst_hbm, zeros_hbm, out_hbm, *refs):
    src_v = refs[0:NBUF]
    dst_v = refs[NBUF:2 * NBUF]
    rows_v = refs[2 * NBUF:3 * NBUF]
    acc_sh = refs[3 * NBUF]
    sem_i = refs[3 * NBUF + 1]
    sems = refs[3 * NBUF + 2:]
    c = lax.axis_index("c")
    s = lax.axis_index("s")
    base0 = s * ROWS_PER_SUB
    pltpu.sync_copy(zeros_hbm.at[pl.ds(base0, ROWS_PER_SUB)],
                    acc_sh.at[pl.ds(base0, ROWS_PER_SUB)])
    plsc.subcore_barrier()

    def make_body(base):
        def body(k, _):
            j0 = base + NBUF * k
            idx_cps = []
            for b in range(NBUF):
                idx_cps.append(
                    pltpu.async_copy(src_hbm.at[j0 + b], src_v[b], sem_i))
                idx_cps.append(
                    pltpu.async_copy(dst_hbm.at[j0 + b], dst_v[b], sem_i))
            for cp in idx_cps:
                cp.wait()
            row_cps = [
                pltpu.async_copy(g_hbm.at[src_v[b]], rows_v[b], sems[b])
                for b in range(NBUF)]
            for b in range(NBUF):
                row_cps[b].wait()
                pltpu.sync_copy(rows_v[b], acc_sh.at[dst_v[b]], add=True)
            return 0
        return body

    @pl.when(c == 0)
    def _():
        lax.fori_loop(0, STEPS_FAST // NBUF,
                      make_body(s * STEPS_FAST), 0)

    @pl.when(c == 1)
    def _():
        lax.fori_loop(0, STEPS_SLOW // NBUF,
                      make_body(CHUNKS_FAST + s * STEPS_SLOW), 0)

    plsc.subcore_barrier()
    base = s * ROWS_PER_SUB
    pltpu.sync_copy(acc_sh.at[pl.ds(base, ROWS_PER_SUB)],
                    out_hbm.at[c, pl.ds(base, ROWS_PER_SUB)])


BLK = 512
GRID = N_PAD // BLK

_row_spec = pl.BlockSpec((BLK, NHID), lambda i: (i, 0))
_par_spec = pl.BlockSpec((NC, BLK, 16), lambda i: (0, i, 0))
_parh_spec = pl.BlockSpec((NC, BLK, NHID), lambda i: (0, i, 0))
_w_spec = pl.BlockSpec((NHID, NHID), lambda i: (0, 0))
_b_spec = pl.BlockSpec((1, NHID), lambda i: (0, 0))


def _tc_prep_body(hp_ref, win_ref, bin_ref, degp_ref,
                  dinv_ref, h0_ref, g_ref):
    i = pl.program_id(0)
    cnt = (jnp.sum(degp_ref[0], axis=1, keepdims=True)
           + jnp.sum(degp_ref[1], axis=1, keepdims=True)) * (1.0 / 16.0)
    deg = cnt + 1.0
    rows = lax.broadcasted_iota(jnp.int32, (BLK, 1), 0) + i * BLK
    dinv = jnp.where(rows < N_NODES, lax.rsqrt(deg), 0.0)
    h1 = jnp.dot(hp_ref[...], win_ref[...],
                 preferred_element_type=jnp.float32) + bin_ref[...]
    h1 = jnp.maximum(h1, 0.0)
    dinv_ref[...] = jnp.broadcast_to(dinv, (BLK, NHID))
    h0_ref[...] = h1
    g_ref[...] = dinv * h1


def _tc_prep(h_pad, w_in, b_in, degp):
    return pl.pallas_call(
        _tc_prep_body,
        grid=(GRID,),
        in_specs=[_row_spec, _w_spec, _b_spec, _par_spec],
        out_specs=[_row_spec, _row_spec, _row_spec],
        out_shape=[jax.ShapeDtypeStruct((N_PAD, NHID), jnp.float32)] * 3,
    )(h_pad, w_in, b_in, degp)


def _make_combine(beta, relu):
    def body(p_ref, g_ref, h0_ref, dinv_ref, w_ref, gout_ref):
        ah = dinv_ref[...] * (p_ref[0] + p_ref[1] + g_ref[...])
        sup = (1.0 - ALPHA) * ah + ALPHA * h0_ref[...]
        h = (1.0 - beta) * sup + beta * jnp.dot(
            sup, w_ref[...], preferred_element_type=jnp.float32)
        if relu:
            h = jnp.maximum(h, 0.0)
        gout_ref[...] = dinv_ref[...] * h

    def run(p, g, h0, dinv2d, w):
        return pl.pallas_call(
            body,
            grid=(GRID,),
            in_specs=[_parh_spec, _row_spec, _row_spec, _row_spec, _w_spec],
            out_specs=_row_spec,
            out_shape=jax.ShapeDtypeStruct((N_PAD, NHID), jnp.float32),
        )(p, g, h0, dinv2d, w)
    return run


def _make_final(beta):
    def body(p_ref, g_ref, h0_ref, dinv_ref, w_ref, wout_ref, bout_ref,
             out_ref):
        ah = dinv_ref[...] * (p_ref[0] + p_ref[1] + g_ref[...])
        sup = (1.0 - ALPHA) * ah + ALPHA * h0_ref[...]
        h = (1.0 - beta) * sup + beta * jnp.dot(
            sup, w_ref[...], preferred_element_type=jnp.float32)
        logits = jnp.dot(h, wout_ref[...],
                         preferred_element_type=jnp.float32) + bout_ref[...]
        col = lax.broadcasted_iota(jnp.int32, (BLK, NHID), 1)
        logits = jnp.where(col < NCLASS, logits, -1e30)
        m = jnp.max(logits, axis=1, keepdims=True)
        lse = jnp.log(jnp.sum(jnp.exp(logits - m), axis=1, keepdims=True)) + m
        out_ref[...] = logits - lse

    def run(p, g, h0, dinv2d, w, w_out, b_out):
        return pl.pallas_call(
            body,
            grid=(GRID,),
            in_specs=[_parh_spec, _row_spec, _row_spec, _row_spec, _w_spec,
                      _w_spec, _b_spec],
            out_specs=_row_spec,
            out_shape=jax.ShapeDtypeStruct((N_PAD, NHID), jnp.float32),
        )(p, g, h0, dinv2d, w, w_out, b_out)
    return run


@jax.jit
def kernel(graph, h, W_in, b_in, W_convs, W_out, b_out):
    graph = graph.astype(jnp.int32)
    pad = jnp.full((E_PAD - N_EDGES,), N_NODES, jnp.int32)
    src = jnp.concatenate([graph[0], pad]).reshape(N_CHUNKS, CHUNK)
    dst = jnp.concatenate([graph[1], pad]).reshape(N_CHUNKS, CHUNK)

    h_pad = jnp.pad(h, ((0, N_PAD - N_NODES), (0, 0)))
    w_out_pad = jnp.pad(W_out, ((0, 0), (0, NHID - NCLASS)))
    b_out_pad = jnp.pad(b_out, ((0, NHID - NCLASS),)).reshape(1, NHID)
    b_in2 = b_in.reshape(1, NHID)

    zeros16 = jnp.zeros((N_PAD, 16), jnp.float32)
    zerosh = jnp.zeros((N_PAD, NHID), jnp.float32)
    degp = _sc_degree(dst, zeros16)
    dinv2d, h0, g = _tc_prep(h_pad, W_in, b_in2, degp)

    for l in range(1, NUM_LAYERS + 1):
        beta = math.log(LAM / l + 1.0)
        p = _sc_propagate(g, src, dst, zerosh)
        if l < NUM_LAYERS:
            g = _make_combine(beta, relu=(l == 1))(p, g, h0, dinv2d,
                                                   W_convs[l - 1])
        else:
            out = _make_final(beta)(p, g, h0, dinv2d, W_convs[l - 1],
                                    w_out_pad, b_out_pad)
    return out[:N_NODES, :NCLASS]
